# baseline (speedup 1.0000x reference)
import functools

import jax
import jax.numpy as jnp
from jax import lax
from jax.experimental import pallas as pl
from jax.experimental.pallas import tpu as pltpu
from jax.experimental.pallas import tpu_sc as plsc

DIM = 64
BLK = 128
L = 16


@functools.cache
def _make_gather(B, S, V, nw, nc):
    assert B == nw * BLK and S % 2 == 0
    mesh = plsc.VectorSubcoreMesh(core_axis_name="c", subcore_axis_name="s")

    @functools.partial(
        pl.kernel,
        mesh=mesh,
        compiler_params=pltpu.CompilerParams(
            use_tc_tiling_on_sc=True, needs_layout_passes=False
        ),
        out_type=jax.ShapeDtypeStruct((S, DIM, B), jnp.float32),
        scratch_types=[
            pltpu.VMEM((DIM * V,), jnp.float32),
            pltpu.VMEM((S * BLK,), jnp.int32),
            pltpu.VMEM((2, DIM, BLK), jnp.float32),
            pltpu.SemaphoreType.DMA((2,)),
        ],
    )
    def gather_kernel(tablet_hbm, idx_hbm, out_hbm, tablet_v, idx_v, slab_v, ssem):
        wid = lax.axis_index("s") * nc + lax.axis_index("c")
        pltpu.sync_copy(tablet_hbm, tablet_v)
        pltpu.sync_copy(idx_hbm.at[pl.ds(wid * S * BLK, S * BLK)], idx_v)

        def body(jj, carry):
            for b in range(2):
                r = jj * 2 + b

                @pl.when(jj > 0)
                def _():
                    pltpu.make_async_copy(
                        slab_v.at[b], out_hbm.at[0, :, pl.ds(0, BLK)], ssem.at[b]
                    ).wait()

                idxv = [
                    idx_v[pl.ds(r * BLK + L * k, L)] for k in range(BLK // L)
                ]

                @plsc.parallel_loop(0, DIM, unroll=16)
                def _(c):
                    for k in range(BLK // L):
                        val = plsc.load_gather(tablet_v, [idxv[k] + c * V])
                        slab_v[b, c, pl.ds(L * k, L)] = val

                pltpu.async_copy(
                    slab_v.at[b],
                    out_hbm.at[r, :, pl.ds(wid * BLK, BLK)],
                    ssem.at[b],
                )
            return carry

        lax.fori_loop(0, S // 2, body, 0)
        for b in range(2):
            pltpu.make_async_copy(
                slab_v.at[b], out_hbm.at[0, :, pl.ds(0, BLK)], ssem.at[b]
            ).wait()

    return gather_kernel


def kernel(x, table_en, table_zh, table_jp):
    table = jnp.concatenate([table_en, table_zh, table_jp], axis=0)
    info = plsc.get_sparse_core_info()
    nw = info.num_cores * info.num_subcores
    B, S = x.shape
    idx1 = x.T.reshape(S, nw, BLK).transpose(1, 0, 2).reshape(-1)
    out_t = _make_gather(B, S, table.shape[0], nw, info.num_cores)(
        table.T.reshape(-1), idx1
    )
    return jnp.transpose(out_t, (2, 0, 1))


# speedup vs baseline: 1.0041x; 1.0041x over previous
import functools

import jax
import jax.numpy as jnp
from jax import lax
from jax.experimental import pallas as pl
from jax.experimental.pallas import tpu as pltpu
from jax.experimental.pallas import tpu_sc as plsc

DIM = 64
BLK = 128
L = 16


@functools.cache
def _make_gather(B, S, V, nw, nc):
    assert B == nw * BLK and S % 2 == 0
    mesh = plsc.VectorSubcoreMesh(core_axis_name="c", subcore_axis_name="s")

    @functools.partial(
        pl.kernel,
        mesh=mesh,
        compiler_params=pltpu.CompilerParams(
            use_tc_tiling_on_sc=True, needs_layout_passes=False
        ),
        out_type=jax.ShapeDtypeStruct((S, DIM, B), jnp.float32),
        scratch_types=[
            pltpu.VMEM((DIM * V,), jnp.float32),
            pltpu.VMEM((S * BLK,), jnp.int32),
            pltpu.VMEM((2, DIM, BLK), jnp.float32),
            pltpu.SemaphoreType.DMA((2,)),
        ],
    )
    def gather_kernel(tablet_hbm, idx_hbm, out_hbm, tablet_v, idx_v, slab_v, ssem):
        wid = lax.axis_index("s") * nc + lax.axis_index("c")
        pltpu.sync_copy(tablet_hbm, tablet_v)
        pltpu.sync_copy(idx_hbm.at[pl.ds(wid * S * BLK, S * BLK)], idx_v)

        def body(jj, carry):
            for b in range(2):
                r = jj * 2 + b

                @pl.when(jj > 0)
                def _():
                    pltpu.make_async_copy(
                        slab_v.at[b], out_hbm.at[0, :, pl.ds(0, BLK)], ssem.at[b]
                    ).wait()

                idxv = [
                    idx_v[pl.ds(r * BLK + L * k, L)] for k in range(BLK // L)
                ]

                @plsc.parallel_loop(0, DIM, unroll=8)
                def _(c):
                    for k in range(BLK // L):
                        val = plsc.load_gather(tablet_v, [idxv[k] + c * V])
                        slab_v[b, c, pl.ds(L * k, L)] = val

                pltpu.async_copy(
                    slab_v.at[b],
                    out_hbm.at[r, :, pl.ds(wid * BLK, BLK)],
                    ssem.at[b],
                )
            return carry

        lax.fori_loop(0, S // 2, body, 0)
        for b in range(2):
            pltpu.make_async_copy(
                slab_v.at[b], out_hbm.at[0, :, pl.ds(0, BLK)], ssem.at[b]
            ).wait()

    return gather_kernel


def kernel(x, table_en, table_zh, table_jp):
    table = jnp.concatenate([table_en, table_zh, table_jp], axis=0)
    info = plsc.get_sparse_core_info()
    nw = info.num_cores * info.num_subcores
    B, S = x.shape
    idx1 = x.T.reshape(S, nw, BLK).transpose(1, 0, 2).reshape(-1)
    out_t = _make_gather(B, S, table.shape[0], nw, info.num_cores)(
        table.T.reshape(-1), idx1
    )
    return jnp.transpose(out_t, (2, 0, 1))


# speedup vs baseline: 1.0186x; 1.0144x over previous
import functools

import jax
import jax.numpy as jnp
from jax import lax
from jax.experimental import pallas as pl
from jax.experimental.pallas import tpu as pltpu
from jax.experimental.pallas import tpu_sc as plsc

DIM = 64
BLK = 128
L = 16


@functools.cache
def _make_gather(B, S, V, nw, nc):
    assert B == nw * BLK and S % 2 == 0
    mesh = plsc.VectorSubcoreMesh(core_axis_name="c", subcore_axis_name="s")

    @functools.partial(
        pl.kernel,
        mesh=mesh,
        compiler_params=pltpu.CompilerParams(
            use_tc_tiling_on_sc=True, needs_layout_passes=False
        ),
        out_type=jax.ShapeDtypeStruct((S, DIM, B), jnp.float32),
        scratch_types=[
            pltpu.VMEM((DIM * V,), jnp.float32),
            pltpu.VMEM((S * BLK,), jnp.int32),
            pltpu.VMEM((2, DIM, BLK), jnp.float32),
            pltpu.SemaphoreType.DMA((2,)),
            pltpu.SemaphoreType.DMA((2,)),
        ],
    )
    def gather_kernel(tablet_hbm, idx_hbm, out_hbm, tablet_v, idx_v, slab_v, ssem, lsem):
        wid = lax.axis_index("s") * nc + lax.axis_index("c")
        pltpu.async_copy(tablet_hbm, tablet_v, lsem.at[0])
        pltpu.async_copy(idx_hbm.at[pl.ds(wid * S * BLK, S * BLK)], idx_v, lsem.at[1])
        pltpu.make_async_copy(tablet_hbm, tablet_v, lsem.at[0]).wait()
        pltpu.make_async_copy(
            idx_hbm.at[pl.ds(0, S * BLK)], idx_v, lsem.at[1]
        ).wait()

        def body(jj, carry):
            for b in range(2):
                r = jj * 2 + b

                @pl.when(jj > 0)
                def _():
                    pltpu.make_async_copy(
                        slab_v.at[b], out_hbm.at[0, :, pl.ds(0, BLK)], ssem.at[b]
                    ).wait()

                idxv = [
                    idx_v[pl.ds(r * BLK + L * k, L)] for k in range(BLK // L)
                ]

                @plsc.parallel_loop(0, DIM, unroll=8)
                def _(c):
                    for k in range(BLK // L):
                        val = plsc.load_gather(tablet_v, [idxv[k] + c * V])
                        slab_v[b, c, pl.ds(L * k, L)] = val

                pltpu.async_copy(
                    slab_v.at[b],
                    out_hbm.at[r, :, pl.ds(wid * BLK, BLK)],
                    ssem.at[b],
                )
            return carry

        lax.fori_loop(0, S // 2, body, 0)
        for b in range(2):
            pltpu.make_async_copy(
                slab_v.at[b], out_hbm.at[0, :, pl.ds(0, BLK)], ssem.at[b]
            ).wait()

    return gather_kernel


def kernel(x, table_en, table_zh, table_jp):
    table = jnp.concatenate([table_en, table_zh, table_jp], axis=0)
    info = plsc.get_sparse_core_info()
    nw = info.num_cores * info.num_subcores
    B, S = x.shape
    idx1 = x.T.reshape(S, nw, BLK).transpose(1, 0, 2).reshape(-1)
    out_t = _make_gather(B, S, table.shape[0], nw, info.num_cores)(
        table.T.reshape(-1), idx1
    )
    return jnp.transpose(out_t, (2, 0, 1))


# speedup vs baseline: 1.0804x; 1.0607x over previous
import functools

import jax
import jax.numpy as jnp
from jax import lax
from jax.experimental import pallas as pl
from jax.experimental.pallas import tpu as pltpu
from jax.experimental.pallas import tpu_sc as plsc

DIM = 64
BLK = 128
L = 16


@functools.cache
def _make_gather(B, S, V, nw, nc):
    assert B == nw * BLK and S % 2 == 0
    mesh = plsc.VectorSubcoreMesh(core_axis_name="c", subcore_axis_name="s")

    @functools.partial(
        pl.kernel,
        mesh=mesh,
        compiler_params=pltpu.CompilerParams(
            use_tc_tiling_on_sc=True, needs_layout_passes=False
        ),
        out_type=jax.ShapeDtypeStruct((S, DIM, B), jnp.float32),
        scratch_types=[
            pltpu.VMEM((DIM * V,), jnp.float32),
            pltpu.VMEM((S, BLK), jnp.int32),
            pltpu.VMEM((2, DIM, BLK), jnp.float32),
            pltpu.SemaphoreType.DMA((2,)),
            pltpu.SemaphoreType.DMA((2,)),
        ],
    )
    def gather_kernel(tablet_hbm, idx_hbm, out_hbm, tablet_v, idx_v, slab_v, ssem, lsem):
        wid = lax.axis_index("s") * nc + lax.axis_index("c")
        pltpu.async_copy(tablet_hbm, tablet_v, lsem.at[0])
        pltpu.async_copy(idx_hbm.at[:, pl.ds(wid * BLK, BLK)], idx_v, lsem.at[1])
        pltpu.make_async_copy(tablet_hbm, tablet_v, lsem.at[0]).wait()
        pltpu.make_async_copy(
            idx_hbm.at[:, pl.ds(0, BLK)], idx_v, lsem.at[1]
        ).wait()

        def body(jj, carry):
            for b in range(2):
                r = jj * 2 + b

                @pl.when(jj > 0)
                def _():
                    pltpu.make_async_copy(
                        slab_v.at[b], out_hbm.at[0, :, pl.ds(0, BLK)], ssem.at[b]
                    ).wait()

                idxv = [
                    idx_v[r, pl.ds(L * k, L)] for k in range(BLK // L)
                ]

                @plsc.parallel_loop(0, DIM, unroll=8)
                def _(c):
                    for k in range(BLK // L):
                        val = plsc.load_gather(tablet_v, [idxv[k] + c * V])
                        slab_v[b, c, pl.ds(L * k, L)] = val

                pltpu.async_copy(
                    slab_v.at[b],
                    out_hbm.at[r, :, pl.ds(wid * BLK, BLK)],
                    ssem.at[b],
                )
            return carry

        lax.fori_loop(0, S // 2, body, 0)
        for b in range(2):
            pltpu.make_async_copy(
                slab_v.at[b], out_hbm.at[0, :, pl.ds(0, BLK)], ssem.at[b]
            ).wait()

    return gather_kernel


def kernel(x, table_en, table_zh, table_jp):
    table = jnp.concatenate([table_en, table_zh, table_jp], axis=0)
    info = plsc.get_sparse_core_info()
    nw = info.num_cores * info.num_subcores
    B, S = x.shape
    out_t = _make_gather(B, S, table.shape[0], nw, info.num_cores)(
        table.T.reshape(-1), x.T
    )
    return jnp.transpose(out_t, (2, 0, 1))
